# baseline (speedup 1.0000x reference)
import functools

import jax
import jax.numpy as jnp
from jax.experimental import pallas as pl

HIDDEN = 1024
NUM_EXPERTS = 8
TOP_K = 2
TOK_TILE = 128


def _moe_body(x_ref, rw_ref, ew_ref, out_ref):
    e = pl.program_id(1)
    x = x_ref[...]
    logits = jnp.dot(x, rw_ref[...], preferred_element_type=jnp.float32)
    m = jnp.max(logits, axis=-1, keepdims=True)
    ex = jnp.exp(logits - m)
    probs = ex / jnp.sum(ex, axis=-1, keepdims=True)
    iota = jax.lax.broadcasted_iota(jnp.int32, probs.shape, 1)
    m1 = jnp.max(probs, axis=-1, keepdims=True)
    i1 = jnp.min(jnp.where(probs == m1, iota, NUM_EXPERTS), axis=-1, keepdims=True)
    probs2 = jnp.where(iota == i1, -jnp.inf, probs)
    m2 = jnp.max(probs2, axis=-1, keepdims=True)
    i2 = jnp.min(jnp.where(probs2 == m2, iota, NUM_EXPERTS), axis=-1, keepdims=True)
    denom = m1 + m2
    w1 = m1 / denom
    w2 = m2 / denom
    coef = jnp.where(i1 == e, w1, 0.0) + jnp.where(i2 == e, w2, 0.0)
    contrib = coef * jnp.dot(x, ew_ref[0], preferred_element_type=jnp.float32)

    @pl.when(e == 0)
    def _init():
        out_ref[...] = contrib

    @pl.when(e > 0)
    def _acc():
        out_ref[...] += contrib


def kernel(x, router_w, expert_w, expert_b):
    B, S, H = x.shape
    x2d = x.reshape(-1, H)
    T = x2d.shape[0]
    out = pl.pallas_call(
        _moe_body,
        grid=(T // TOK_TILE, NUM_EXPERTS),
        in_specs=[
            pl.BlockSpec((TOK_TILE, H), lambda i, e: (i, 0)),
            pl.BlockSpec((H, NUM_EXPERTS), lambda i, e: (0, 0)),
            pl.BlockSpec((1, H, H), lambda i, e: (e, 0, 0)),
        ],
        out_specs=pl.BlockSpec((TOK_TILE, H), lambda i, e: (i, 0)),
        out_shape=jax.ShapeDtypeStruct((T, H), jnp.float32),
    )(x2d, router_w, expert_w)
    return out.reshape(B, S, H)


# speedup vs baseline: 3.4429x; 3.4429x over previous
import functools

import jax
import jax.numpy as jnp
from jax.experimental import pallas as pl

HIDDEN = 1024
NUM_EXPERTS = 8
TOP_K = 2
TOK_TILE = 128


def _moe_body(x_ref, rw_ref, ew_ref, out_ref):
    e = pl.program_id(0)
    x = x_ref[...]
    logits = jnp.dot(x, rw_ref[...], preferred_element_type=jnp.float32)
    m = jnp.max(logits, axis=-1, keepdims=True)
    ex = jnp.exp(logits - m)
    probs = ex / jnp.sum(ex, axis=-1, keepdims=True)
    iota = jax.lax.broadcasted_iota(jnp.int32, probs.shape, 1)
    m1 = jnp.max(probs, axis=-1, keepdims=True)
    i1 = jnp.min(jnp.where(probs == m1, iota, NUM_EXPERTS), axis=-1, keepdims=True)
    probs2 = jnp.where(iota == i1, -jnp.inf, probs)
    m2 = jnp.max(probs2, axis=-1, keepdims=True)
    i2 = jnp.min(jnp.where(probs2 == m2, iota, NUM_EXPERTS), axis=-1, keepdims=True)
    denom = m1 + m2
    w1 = m1 / denom
    w2 = m2 / denom
    coef = jnp.where(i1 == e, w1, 0.0) + jnp.where(i2 == e, w2, 0.0)
    contrib = coef * jnp.dot(x, ew_ref[0], preferred_element_type=jnp.float32)

    @pl.when(e == 0)
    def _init():
        out_ref[...] = contrib

    @pl.when(e > 0)
    def _acc():
        out_ref[...] += contrib


def kernel(x, router_w, expert_w, expert_b):
    B, S, H = x.shape
    x2d = x.reshape(-1, H)
    T = x2d.shape[0]
    out = pl.pallas_call(
        _moe_body,
        grid=(NUM_EXPERTS,),
        in_specs=[
            pl.BlockSpec((T, H), lambda e: (0, 0)),
            pl.BlockSpec((H, NUM_EXPERTS), lambda e: (0, 0)),
            pl.BlockSpec((1, H, H), lambda e: (e, 0, 0)),
        ],
        out_specs=pl.BlockSpec((T, H), lambda e: (0, 0)),
        out_shape=jax.ShapeDtypeStruct((T, H), jnp.float32),
    )(x2d, router_w, expert_w)
    return out.reshape(B, S, H)
